# Initial kernel scaffold; baseline (speedup 1.0000x reference)
import jax
import jax.numpy as jnp
from jax.experimental import pallas as pl


def kernel(claims, ground_truths, agent_ids, trust_logits):
    raise NotImplementedError("write your pallas kernel here")


# speedup vs baseline: 1.1645x; 1.1645x over previous
import functools

import jax
import jax.numpy as jnp
from jax import lax
from jax.experimental import pallas as pl
from jax.experimental.pallas import tpu as pltpu
from jax.experimental.pallas import tpu_sc as plsc

_BATCH = 16384
_NC = 2
_NS = 16
_L = 16
_NW = _NC * _NS
_BPW = _BATCH // _NW
_CHUNK = 128
_NCHUNK = _BPW // _CHUNK
_NVEC = _BPW // _L


def _make_sc_kernel():
    mesh = plsc.VectorSubcoreMesh(core_axis_name="c", subcore_axis_name="s")

    @functools.partial(
        pl.kernel,
        mesh=mesh,
        out_type=jax.ShapeDtypeStruct((_NW, _L), jnp.float32),
        scratch_types=[
            pltpu.VMEM((_NCHUNK, _CHUNK), jnp.int32),
            pltpu.VMEM((_NCHUNK, _CHUNK), jnp.float32),
            pltpu.VMEM((_BPW,), jnp.float32),
            pltpu.VMEM((_BPW,), jnp.float32),
            pltpu.VMEM((_L,), jnp.float32),
            pltpu.SemaphoreType.DMA,
        ],
    )
    def trust_loss(ids_hbm, logits_hbm, claims_hbm, gt_hbm, out_hbm,
                   idx_v, gath_v, c_v, g_v, acc_v, sem):
        wid = lax.axis_index("s") * _NC + lax.axis_index("c")
        pltpu.sync_copy(ids_hbm.at[wid], idx_v)
        copies = [
            pltpu.async_copy(logits_hbm.at[idx_v.at[j]], gath_v.at[j], sem)
            for j in range(_NCHUNK)
        ]
        pltpu.sync_copy(claims_hbm.at[wid], c_v)
        pltpu.sync_copy(gt_hbm.at[wid], g_v)
        for cp in copies:
            cp.wait()

        acc = jnp.zeros((_L,), jnp.float32)
        for v in range(_NVEC):
            j, off = divmod(v * _L, _CHUNK)
            x = gath_v[j, pl.ds(off, _L)]
            c = c_v[pl.ds(v * _L, _L)]
            g = g_v[pl.ds(v * _L, _L)]
            t = 1.0 / (1.0 + jnp.exp(-x))
            e = t * (c - 0.5) + (0.5 - g)
            acc = acc + e * e + (0.3 * t) * jnp.abs(c - g)
        acc_v[...] = acc
        pltpu.sync_copy(acc_v, out_hbm.at[wid])

    return trust_loss


_sc_trust_loss = _make_sc_kernel()


def kernel(claims, ground_truths, agent_ids, trust_logits):
    ids = agent_ids.astype(jnp.int32).reshape(_NW, _NCHUNK, _CHUNK)
    c = claims.reshape(_NW, _BPW)
    g = ground_truths.reshape(_NW, _BPW)
    partials = _sc_trust_loss(ids, trust_logits, c, g)
    return jnp.sum(partials) * (1.0 / _BATCH)
